# baseline (speedup 1.0000x reference)
import jax
import jax.numpy as jnp
from jax import lax
from jax.experimental import pallas as pl
from jax.experimental.pallas import tpu as pltpu


_COL_START = 26
_COL_COUNT = 100
_ALIGNED_START = 0
_SHIFT = _COL_START - _ALIGNED_START
_READ_W = 126
_BLOCK = 2048
_GRID = 16384 // _BLOCK


def _in_copy(in_hbm, ibuf, isem, i, slot):
    return pltpu.make_async_copy(
        in_hbm.at[pl.ds(i * _BLOCK, _BLOCK), pl.ds(_ALIGNED_START, _READ_W)],
        ibuf.at[slot],
        isem.at[slot],
    )


def _out_copy(out_hbm, obuf, osem, i, slot):
    return pltpu.make_async_copy(
        obuf.at[slot],
        out_hbm.at[pl.ds(i * _BLOCK, _BLOCK), :],
        osem.at[slot],
    )


def _body(in_hbm, out_hbm, ibuf, obuf, isem, osem):
    i = pl.program_id(0)
    slot = lax.rem(i, 2)

    @pl.when(i == 0)
    def _():
        _in_copy(in_hbm, ibuf, isem, 0, 0).start()

    @pl.when(i + 1 < _GRID)
    def _():
        _in_copy(in_hbm, ibuf, isem, i + 1, lax.rem(i + 1, 2)).start()

    _in_copy(in_hbm, ibuf, isem, i, slot).wait()

    @pl.when(i >= 2)
    def _():
        _out_copy(out_hbm, obuf, osem, i - 2, slot).wait()

    obuf[slot] = ibuf[slot][:, _SHIFT:_SHIFT + _COL_COUNT]
    _out_copy(out_hbm, obuf, osem, i, slot).start()

    @pl.when(i == _GRID - 1)
    def _():
        _out_copy(out_hbm, obuf, osem, i - 1, lax.rem(i + 1, 2)).wait()
        _out_copy(out_hbm, obuf, osem, i, slot).wait()


def kernel(inputs):
    n_rows, _ = inputs.shape
    return pl.pallas_call(
        _body,
        grid=(_GRID,),
        in_specs=[pl.BlockSpec(memory_space=pltpu.MemorySpace.HBM)],
        out_specs=pl.BlockSpec(memory_space=pltpu.MemorySpace.HBM),
        out_shape=jax.ShapeDtypeStruct((n_rows, _COL_COUNT), jnp.float32),
        scratch_shapes=[
            pltpu.VMEM((2, _BLOCK, _READ_W), jnp.float32),
            pltpu.VMEM((2, _BLOCK, _COL_COUNT), jnp.float32),
            pltpu.SemaphoreType.DMA((2,)),
            pltpu.SemaphoreType.DMA((2,)),
        ],
    )(inputs)


# speedup vs baseline: 1.1032x; 1.1032x over previous
import jax
import jax.numpy as jnp
from jax import lax
from jax.experimental import pallas as pl
from jax.experimental.pallas import tpu as pltpu


_COL_START = 26
_COL_COUNT = 100
_ALIGNED_START = 0
_SHIFT = _COL_START - _ALIGNED_START
_READ_W = 126
_BLOCK = 4096
_GRID = 16384 // _BLOCK


def _in_copy(in_hbm, ibuf, isem, i, slot):
    return pltpu.make_async_copy(
        in_hbm.at[pl.ds(i * _BLOCK, _BLOCK), pl.ds(_ALIGNED_START, _READ_W)],
        ibuf.at[slot],
        isem.at[slot],
    )


def _out_copy(out_hbm, obuf, osem, i, slot):
    return pltpu.make_async_copy(
        obuf.at[slot],
        out_hbm.at[pl.ds(i * _BLOCK, _BLOCK), :],
        osem.at[slot],
    )


def _body(in_hbm, out_hbm, ibuf, obuf, isem, osem):
    i = pl.program_id(0)
    slot = lax.rem(i, 2)

    @pl.when(i == 0)
    def _():
        _in_copy(in_hbm, ibuf, isem, 0, 0).start()

    @pl.when(i + 1 < _GRID)
    def _():
        _in_copy(in_hbm, ibuf, isem, i + 1, lax.rem(i + 1, 2)).start()

    _in_copy(in_hbm, ibuf, isem, i, slot).wait()

    @pl.when(i >= 2)
    def _():
        _out_copy(out_hbm, obuf, osem, i - 2, slot).wait()

    obuf[slot] = ibuf[slot][:, _SHIFT:_SHIFT + _COL_COUNT]
    _out_copy(out_hbm, obuf, osem, i, slot).start()

    @pl.when(i == _GRID - 1)
    def _():
        _out_copy(out_hbm, obuf, osem, i - 1, lax.rem(i + 1, 2)).wait()
        _out_copy(out_hbm, obuf, osem, i, slot).wait()


def kernel(inputs):
    n_rows, _ = inputs.shape
    return pl.pallas_call(
        _body,
        grid=(_GRID,),
        in_specs=[pl.BlockSpec(memory_space=pltpu.MemorySpace.HBM)],
        out_specs=pl.BlockSpec(memory_space=pltpu.MemorySpace.HBM),
        out_shape=jax.ShapeDtypeStruct((n_rows, _COL_COUNT), jnp.float32),
        scratch_shapes=[
            pltpu.VMEM((2, _BLOCK, _READ_W), jnp.float32),
            pltpu.VMEM((2, _BLOCK, _COL_COUNT), jnp.float32),
            pltpu.SemaphoreType.DMA((2,)),
            pltpu.SemaphoreType.DMA((2,)),
        ],
    )(inputs)


# speedup vs baseline: 1.1987x; 1.0866x over previous
import jax
import jax.numpy as jnp
from jax import lax
from jax.experimental import pallas as pl
from jax.experimental.pallas import tpu as pltpu


_COL_START = 26
_COL_COUNT = 100
_ALIGNED_START = 0
_SHIFT = _COL_START - _ALIGNED_START
_READ_W = 126
_BLOCK = 8192
_GRID = 16384 // _BLOCK


def _in_copy(in_hbm, ibuf, isem, i, slot):
    return pltpu.make_async_copy(
        in_hbm.at[pl.ds(i * _BLOCK, _BLOCK), pl.ds(_ALIGNED_START, _READ_W)],
        ibuf.at[slot],
        isem.at[slot],
    )


def _out_copy(out_hbm, obuf, osem, i, slot):
    return pltpu.make_async_copy(
        obuf.at[slot],
        out_hbm.at[pl.ds(i * _BLOCK, _BLOCK), :],
        osem.at[slot],
    )


def _body(in_hbm, out_hbm, ibuf, obuf, isem, osem):
    i = pl.program_id(0)
    slot = lax.rem(i, 2)

    @pl.when(i == 0)
    def _():
        _in_copy(in_hbm, ibuf, isem, 0, 0).start()

    @pl.when(i + 1 < _GRID)
    def _():
        _in_copy(in_hbm, ibuf, isem, i + 1, lax.rem(i + 1, 2)).start()

    _in_copy(in_hbm, ibuf, isem, i, slot).wait()

    @pl.when(i >= 2)
    def _():
        _out_copy(out_hbm, obuf, osem, i - 2, slot).wait()

    obuf[slot] = ibuf[slot][:, _SHIFT:_SHIFT + _COL_COUNT]
    _out_copy(out_hbm, obuf, osem, i, slot).start()

    @pl.when(i == _GRID - 1)
    def _():
        _out_copy(out_hbm, obuf, osem, i - 1, lax.rem(i + 1, 2)).wait()
        _out_copy(out_hbm, obuf, osem, i, slot).wait()


def kernel(inputs):
    n_rows, _ = inputs.shape
    return pl.pallas_call(
        _body,
        grid=(_GRID,),
        in_specs=[pl.BlockSpec(memory_space=pltpu.MemorySpace.HBM)],
        out_specs=pl.BlockSpec(memory_space=pltpu.MemorySpace.HBM),
        out_shape=jax.ShapeDtypeStruct((n_rows, _COL_COUNT), jnp.float32),
        scratch_shapes=[
            pltpu.VMEM((2, _BLOCK, _READ_W), jnp.float32),
            pltpu.VMEM((2, _BLOCK, _COL_COUNT), jnp.float32),
            pltpu.SemaphoreType.DMA((2,)),
            pltpu.SemaphoreType.DMA((2,)),
        ],
    )(inputs)


# speedup vs baseline: 1.2490x; 1.0420x over previous
import jax
import jax.numpy as jnp
from jax.experimental import pallas as pl
from jax.experimental.pallas import tpu as pltpu


_COL_START = 26
_COL_COUNT = 100
_N_ROWS = 16384
_NBLK = 4
_BLOCK = _N_ROWS // _NBLK


def _body(in_hbm, out_hbm, ibuf, obuf, isem, osem):
    def in_copy(i):
        return pltpu.make_async_copy(
            in_hbm.at[pl.ds(i * _BLOCK, _BLOCK), :],
            ibuf.at[i],
            isem.at[i],
        )

    def out_copy(i):
        return pltpu.make_async_copy(
            obuf.at[i],
            out_hbm.at[pl.ds(i * _BLOCK, _BLOCK), :],
            osem.at[i],
        )

    for i in range(_NBLK):
        in_copy(i).start()
    for i in range(_NBLK):
        in_copy(i).wait()
        obuf[i] = ibuf[i][:, _COL_START:_COL_START + _COL_COUNT]
        out_copy(i).start()
    for i in range(_NBLK):
        out_copy(i).wait()


def kernel(inputs):
    n_rows, n_cols = inputs.shape
    return pl.pallas_call(
        _body,
        in_specs=[pl.BlockSpec(memory_space=pltpu.MemorySpace.HBM)],
        out_specs=pl.BlockSpec(memory_space=pltpu.MemorySpace.HBM),
        out_shape=jax.ShapeDtypeStruct((n_rows, _COL_COUNT), jnp.float32),
        scratch_shapes=[
            pltpu.VMEM((_NBLK, _BLOCK, 126), jnp.float32),
            pltpu.VMEM((_NBLK, _BLOCK, _COL_COUNT), jnp.float32),
            pltpu.SemaphoreType.DMA((_NBLK,)),
            pltpu.SemaphoreType.DMA((_NBLK,)),
        ],
    )(inputs)


# speedup vs baseline: 1.2546x; 1.0045x over previous
import jax
import jax.numpy as jnp
from jax.experimental import pallas as pl
from jax.experimental.pallas import tpu as pltpu


_COL_START = 26
_COL_COUNT = 100
_N_ROWS = 16384
_NBLK = 8
_BLOCK = _N_ROWS // _NBLK


def _body(in_hbm, out_hbm, ibuf, obuf, isem, osem):
    def in_copy(i):
        return pltpu.make_async_copy(
            in_hbm.at[pl.ds(i * _BLOCK, _BLOCK), :],
            ibuf.at[i],
            isem.at[i],
        )

    def out_copy(i):
        return pltpu.make_async_copy(
            obuf.at[i],
            out_hbm.at[pl.ds(i * _BLOCK, _BLOCK), :],
            osem.at[i],
        )

    for i in range(_NBLK):
        in_copy(i).start()
    for i in range(_NBLK):
        in_copy(i).wait()
        obuf[i] = ibuf[i][:, _COL_START:_COL_START + _COL_COUNT]
        out_copy(i).start()
    for i in range(_NBLK):
        out_copy(i).wait()


def kernel(inputs):
    n_rows, n_cols = inputs.shape
    return pl.pallas_call(
        _body,
        in_specs=[pl.BlockSpec(memory_space=pltpu.MemorySpace.HBM)],
        out_specs=pl.BlockSpec(memory_space=pltpu.MemorySpace.HBM),
        out_shape=jax.ShapeDtypeStruct((n_rows, _COL_COUNT), jnp.float32),
        scratch_shapes=[
            pltpu.VMEM((_NBLK, _BLOCK, 126), jnp.float32),
            pltpu.VMEM((_NBLK, _BLOCK, _COL_COUNT), jnp.float32),
            pltpu.SemaphoreType.DMA((_NBLK,)),
            pltpu.SemaphoreType.DMA((_NBLK,)),
        ],
    )(inputs)
